# baseline (speedup 1.0000x reference)
import functools

import jax
import jax.numpy as jnp
from jax import lax
from jax.experimental import pallas as pl
from jax.experimental.pallas import tpu as pltpu
from jax.experimental.pallas import tpu_sc as plsc

B, L, D, V = 32, 4096, 256, 5
N = B * L

_SC_INFO = plsc.get_sparse_core_info()
NC = _SC_INFO.num_cores
NS = _SC_INFO.num_subcores
NW = NC * NS
TOK_PER_W = N // NW
CT = 256
NCHUNK = TOK_PER_W // CT


def _ln_body(tab_ref, gamma_ref, beta_ref, out_ref):
    tab = tab_ref[...]
    mean = jnp.mean(tab, axis=1, keepdims=True)
    var = jnp.mean((tab - mean) ** 2, axis=1, keepdims=True)
    ntab = (tab - mean) * jax.lax.rsqrt(var + 1e-12)
    out_ref[...] = ntab * gamma_ref[...] + beta_ref[...]


def _normed_table(emb_table, ln_gamma, ln_beta):
    return pl.pallas_call(
        _ln_body,
        out_shape=jax.ShapeDtypeStruct((V, D), jnp.float32),
    )(emb_table, ln_gamma.reshape(1, D), ln_beta.reshape(1, D))


@functools.partial(
    pl.kernel,
    out_type=jax.ShapeDtypeStruct((N, D), jnp.float32),
    mesh=plsc.VectorSubcoreMesh(core_axis_name="c", subcore_axis_name="s"),
    scratch_types=[
        pltpu.VMEM((CT,), jnp.int32),
        pltpu.VMEM((CT, D), jnp.float32),
        pltpu.SemaphoreType.DMA,
    ],
)
def _sc_gather(ntab_hbm, ids_hbm, out_hbm, idx_v, rows_v, sem):
    wid = lax.axis_index("s") * NC + lax.axis_index("c")
    base = wid * TOK_PER_W

    def chunk(g, carry):
        off = base + g * CT
        pltpu.sync_copy(ids_hbm.at[pl.ds(off, CT)], idx_v)
        pltpu.async_copy(ntab_hbm.at[idx_v], rows_v, sem).wait()
        pltpu.sync_copy(rows_v, out_hbm.at[pl.ds(off, CT)])
        return carry

    lax.fori_loop(0, NCHUNK, chunk, 0)


@functools.partial(jax.jit, static_argnames=())
def kernel(snp_ids, is_padding, emb_table, ln_gamma, ln_beta):
    ntab = _normed_table(emb_table, ln_gamma, ln_beta)
    out = _sc_gather(ntab, snp_ids.reshape(N))
    return out.reshape(B, L, D), is_padding


# speedup vs baseline: 1.0007x; 1.0007x over previous
import functools

import jax
import jax.numpy as jnp
from jax import lax
from jax.experimental import pallas as pl
from jax.experimental.pallas import tpu as pltpu
from jax.experimental.pallas import tpu_sc as plsc

B, L, D, V = 32, 4096, 256, 5
N = B * L

_SC_INFO = plsc.get_sparse_core_info()
NC = _SC_INFO.num_cores
NS = _SC_INFO.num_subcores
NW = NC * NS
TOK_PER_W = N // NW
CT = 128
NCHUNK = TOK_PER_W // CT


def _ln_body(tab_ref, gamma_ref, beta_ref, out_ref):
    tab = tab_ref[...]
    mean = jnp.mean(tab, axis=1, keepdims=True)
    var = jnp.mean((tab - mean) ** 2, axis=1, keepdims=True)
    ntab = (tab - mean) * jax.lax.rsqrt(var + 1e-12)
    out_ref[...] = ntab * gamma_ref[...] + beta_ref[...]


def _normed_table(emb_table, ln_gamma, ln_beta):
    return pl.pallas_call(
        _ln_body,
        out_shape=jax.ShapeDtypeStruct((V, D), jnp.float32),
    )(emb_table, ln_gamma.reshape(1, D), ln_beta.reshape(1, D))


@functools.partial(
    pl.kernel,
    out_type=jax.ShapeDtypeStruct((N, D), jnp.float32),
    mesh=plsc.VectorSubcoreMesh(core_axis_name="c", subcore_axis_name="s"),
    scratch_types=[
        pltpu.VMEM((TOK_PER_W,), jnp.int32),
        pltpu.VMEM((CT, D), jnp.float32),
        pltpu.VMEM((CT, D), jnp.float32),
        pltpu.SemaphoreType.DMA,
        pltpu.SemaphoreType.DMA,
        pltpu.SemaphoreType.DMA,
        pltpu.SemaphoreType.DMA,
    ],
)
def _sc_gather(ntab_hbm, ids_hbm, out_hbm, idx_all, rows0, rows1,
               gsem0, gsem1, osem0, osem1):
    wid = lax.axis_index("s") * NC + lax.axis_index("c")
    base = wid * TOK_PER_W

    pltpu.sync_copy(ids_hbm.at[pl.ds(base, TOK_PER_W)], idx_all)

    def gather(g, rows, sem):
        src = ntab_hbm.at[idx_all.at[pl.ds(g * CT, CT)]]
        pltpu.make_async_copy(src, rows, sem).start()

    def gather_wait(g, rows, sem):
        src = ntab_hbm.at[idx_all.at[pl.ds(g * CT, CT)]]
        pltpu.make_async_copy(src, rows, sem).wait()

    def put(g, rows, sem):
        pltpu.make_async_copy(rows, out_hbm.at[pl.ds(base + g * CT, CT)],
                              sem).start()

    def put_wait(g, rows, sem):
        pltpu.make_async_copy(rows, out_hbm.at[pl.ds(base + g * CT, CT)],
                              sem).wait()

    gather(0, rows0, gsem0)
    npairs = NCHUNK // 2

    def pair(g2, carry):
        g = g2 * 2

        @pl.when(g2 > 0)
        def _():
            put_wait(g - 1, rows1, osem1)

        gather(g + 1, rows1, gsem1)
        gather_wait(g, rows0, gsem0)
        put(g, rows0, osem0)
        gather_wait(g + 1, rows1, gsem1)
        put_wait(g, rows0, osem0)
        put(g + 1, rows1, osem1)

        @pl.when(g2 + 1 < npairs)
        def _():
            gather(g + 2, rows0, gsem0)

        return carry

    lax.fori_loop(0, npairs, pair, 0)
    put_wait(NCHUNK - 1, rows1, osem1)


@functools.partial(jax.jit, static_argnames=())
def kernel(snp_ids, is_padding, emb_table, ln_gamma, ln_beta):
    ntab = _normed_table(emb_table, ln_gamma, ln_beta)
    out = _sc_gather(ntab, snp_ids.reshape(N))
    return out.reshape(B, L, D), is_padding


# speedup vs baseline: 9.1229x; 9.1166x over previous
import functools

import jax
import jax.numpy as jnp
from jax import lax
from jax.experimental import pallas as pl
from jax.experimental.pallas import tpu as pltpu
from jax.experimental.pallas import tpu_sc as plsc

B, L, D, V = 32, 4096, 256, 5
N = B * L

_SC_INFO = plsc.get_sparse_core_info()
NC = _SC_INFO.num_cores
NS = _SC_INFO.num_subcores
NW = NC * NS
TOK_PER_W = N // NW
CT = 128
NCHUNK = TOK_PER_W // CT
REP = 1024


def _ln_body(tab_ref, gamma_ref, beta_ref, out_ref):
    tab = tab_ref[...]
    mean = jnp.mean(tab, axis=1, keepdims=True)
    var = jnp.mean((tab - mean) ** 2, axis=1, keepdims=True)
    ntab = (tab - mean) * jax.lax.rsqrt(var + 1e-12)
    out_ref[...] = ntab * gamma_ref[...] + beta_ref[...]


def _normed_table(emb_table, ln_gamma, ln_beta):
    return pl.pallas_call(
        _ln_body,
        out_shape=jax.ShapeDtypeStruct((V, D), jnp.float32),
    )(emb_table, ln_gamma.reshape(1, D), ln_beta.reshape(1, D))


@functools.partial(
    pl.kernel,
    out_type=jax.ShapeDtypeStruct((N, D), jnp.float32),
    mesh=plsc.VectorSubcoreMesh(core_axis_name="c", subcore_axis_name="s"),
    scratch_types=[
        pltpu.VMEM((TOK_PER_W,), jnp.int32),
        pltpu.VMEM((CT, D), jnp.float32),
        pltpu.VMEM((CT, D), jnp.float32),
        pltpu.SemaphoreType.DMA,
        pltpu.SemaphoreType.DMA,
        pltpu.SemaphoreType.DMA,
        pltpu.SemaphoreType.DMA,
    ],
)
def _sc_gather(ntab_hbm, ids_hbm, out_hbm, idx_all, rows0, rows1,
               gsem0, gsem1, osem0, osem1):
    wid = lax.axis_index("s") * NC + lax.axis_index("c")
    base = wid * TOK_PER_W

    pltpu.sync_copy(ids_hbm.at[pl.ds(base, TOK_PER_W)], idx_all)

    iota = lax.iota(jnp.int32, 16)

    def remap(i, carry):
        sl = pl.ds(i * 16, 16)
        rep = jnp.bitwise_and(i, (REP * 16) // 256 - 1)
        idx_all[sl] = idx_all[sl] + V * iota + (V * 16) * rep
        return carry

    lax.fori_loop(0, TOK_PER_W // 16, remap, 0)

    def gather(g, rows, sem):
        src = ntab_hbm.at[idx_all.at[pl.ds(g * CT, CT)]]
        pltpu.make_async_copy(src, rows, sem).start()

    def gather_wait(g, rows, sem):
        src = ntab_hbm.at[idx_all.at[pl.ds(g * CT, CT)]]
        pltpu.make_async_copy(src, rows, sem).wait()

    def put(g, rows, sem):
        pltpu.make_async_copy(rows, out_hbm.at[pl.ds(base + g * CT, CT)],
                              sem).start()

    def put_wait(g, rows, sem):
        pltpu.make_async_copy(rows, out_hbm.at[pl.ds(base + g * CT, CT)],
                              sem).wait()

    gather(0, rows0, gsem0)
    npairs = NCHUNK // 2

    def pair(g2, carry):
        g = g2 * 2

        @pl.when(g2 > 0)
        def _():
            put_wait(g - 1, rows1, osem1)

        gather(g + 1, rows1, gsem1)
        gather_wait(g, rows0, gsem0)
        put(g, rows0, osem0)
        gather_wait(g + 1, rows1, gsem1)
        put_wait(g, rows0, osem0)
        put(g + 1, rows1, osem1)

        @pl.when(g2 + 1 < npairs)
        def _():
            gather(g + 2, rows0, gsem0)

        return carry

    lax.fori_loop(0, npairs, pair, 0)
    put_wait(NCHUNK - 1, rows1, osem1)


@functools.partial(jax.jit, static_argnames=())
def kernel(snp_ids, is_padding, emb_table, ln_gamma, ln_beta):
    ntab = _normed_table(emb_table, ln_gamma, ln_beta)
    ntab_rep = jnp.tile(ntab, (REP, 1))
    out = _sc_gather(ntab_rep, snp_ids.reshape(N))
    return out.reshape(B, L, D), is_padding
